# baseline (speedup 1.0000x reference)
import functools

import jax
import jax.numpy as jnp
from jax import lax
from jax.experimental import pallas as pl
from jax.experimental.pallas import tpu as pltpu
from jax.experimental.pallas import tpu_sc as plsc

N = 16384
EMBED = 64
GRP = 8
VOCAB = 1000000
NUM_CORES = 2
NUM_SUBCORES = 16
NW = NUM_CORES * NUM_SUBCORES
BPW = N // NW
LANES = 16
NGROUP = BPW // LANES

_mesh = plsc.VectorSubcoreMesh(core_axis_name="c", subcore_axis_name="s")


@functools.partial(
    pl.kernel,
    mesh=_mesh,
    out_type=jax.ShapeDtypeStruct((N, EMBED), jnp.float32),
    scratch_types=[
        pltpu.VMEM((BPW,), jnp.int32),
        pltpu.VMEM((BPW, EMBED), jnp.float32),
        pltpu.SemaphoreType.DMA,
    ],
)
def _gather_kernel(idx_hbm, table_hbm, out_hbm, idx_v, rows_v, sem):
    wid = lax.axis_index("s") * NUM_CORES + lax.axis_index("c")
    base = wid * BPW
    pltpu.sync_copy(idx_hbm.at[pl.ds(base, BPW)], idx_v)

    def group(g, _):
        vec = idx_v[pl.ds(g * LANES, LANES)]
        for k in range(LANES):
            row = vec[k]
            j = g * LANES + k
            pltpu.async_copy(
                table_hbm.at[row // GRP].at[pl.ds(row % GRP, 1)],
                rows_v.at[pl.ds(j, 1)],
                sem,
            )
        return 0

    lax.fori_loop(0, NGROUP, group, 0)
    pltpu.make_async_copy(
        out_hbm.at[pl.ds(0, BPW)], rows_v, sem
    ).wait()
    pltpu.sync_copy(rows_v, out_hbm.at[pl.ds(base, BPW)])


def kernel(data, ivectors):
    table3 = ivectors.reshape(VOCAB // GRP, GRP, EMBED)
    return _gather_kernel(data.astype(jnp.int32), table3)
